# baseline (speedup 1.0000x reference)
import jax
import jax.numpy as jnp
from jax.experimental import pallas as pl
from jax.experimental.pallas import tpu as pltpu


def _mse_onehot_kernel(x_ref, t_ref, out_ref):
    b = pl.program_id(0)
    c = pl.program_id(1)

    @pl.when(jnp.logical_and(b == 0, c == 0))
    def _init():
        out_ref[0] = 0.0

    x = x_ref[0, 0]
    t = t_ref[0]
    mask = (t == c).astype(jnp.float32)
    d = x - mask
    out_ref[0] += jnp.sum(d * d)


def kernel(output, target):
    B, C, H, W = output.shape
    target = target.astype(jnp.int32)

    ssum = pl.pallas_call(
        _mse_onehot_kernel,
        grid=(B, C),
        in_specs=[
            pl.BlockSpec((1, 1, H, W), lambda b, c: (b, c, 0, 0)),
            pl.BlockSpec((1, H, W), lambda b, c: (b, 0, 0)),
        ],
        out_specs=pl.BlockSpec(memory_space=pltpu.SMEM),
        out_shape=jax.ShapeDtypeStruct((1,), jnp.float32),
    )(output, target)

    n = B * C * H * W
    return ssum[0] / jnp.float32(n)


# speedup vs baseline: 1.0210x; 1.0210x over previous
import jax
import jax.numpy as jnp
from jax.experimental import pallas as pl
from jax.experimental.pallas import tpu as pltpu


def _mse_onehot_kernel(x_ref, t_ref, out_ref, acc_ref):
    b = pl.program_id(0)
    c = pl.program_id(1)

    x = x_ref[0, 0]
    t = t_ref[0]
    mask = (t == c).astype(jnp.float32)
    d = x - mask
    d2 = d * d

    @pl.when(jnp.logical_and(b == 0, c == 0))
    def _init():
        acc_ref[...] = d2

    @pl.when(jnp.logical_or(b != 0, c != 0))
    def _accum():
        acc_ref[...] += d2

    nb = pl.num_programs(0)
    nc = pl.num_programs(1)

    @pl.when(jnp.logical_and(b == nb - 1, c == nc - 1))
    def _done():
        out_ref[0] = jnp.sum(acc_ref[...])


def kernel(output, target):
    B, C, H, W = output.shape
    target = target.astype(jnp.int32)

    ssum = pl.pallas_call(
        _mse_onehot_kernel,
        grid=(B, C),
        in_specs=[
            pl.BlockSpec((1, 1, H, W), lambda b, c: (b, c, 0, 0)),
            pl.BlockSpec((1, H, W), lambda b, c: (b, 0, 0)),
        ],
        out_specs=pl.BlockSpec(memory_space=pltpu.SMEM),
        out_shape=jax.ShapeDtypeStruct((1,), jnp.float32),
        scratch_shapes=[pltpu.VMEM((H, W), jnp.float32)],
    )(output, target)

    n = B * C * H * W
    return ssum[0] / jnp.float32(n)


# speedup vs baseline: 1.0760x; 1.0538x over previous
import jax
import jax.numpy as jnp
from jax.experimental import pallas as pl
from jax.experimental.pallas import tpu as pltpu


def _mse_onehot_kernel(x_ref, t_ref, out_ref, acc_ref):
    b = pl.program_id(0)
    c = pl.program_id(1)

    x = x_ref[0, 0]
    t = t_ref[0]
    mask = (t == c).astype(jnp.float32)
    d = x - mask
    d2 = d * d
    H, W = d2.shape
    part = jnp.sum(d2.reshape(H // 8, 8, W), axis=0)

    @pl.when(jnp.logical_and(b == 0, c == 0))
    def _init():
        acc_ref[...] = part

    @pl.when(jnp.logical_or(b != 0, c != 0))
    def _accum():
        acc_ref[...] += part

    nb = pl.num_programs(0)
    nc = pl.num_programs(1)

    @pl.when(jnp.logical_and(b == nb - 1, c == nc - 1))
    def _done():
        out_ref[0] = jnp.sum(acc_ref[...])


def kernel(output, target):
    B, C, H, W = output.shape
    target = target.astype(jnp.int32)

    ssum = pl.pallas_call(
        _mse_onehot_kernel,
        grid=(B, C),
        in_specs=[
            pl.BlockSpec((1, 1, H, W), lambda b, c: (b, c, 0, 0)),
            pl.BlockSpec((1, H, W), lambda b, c: (b, 0, 0)),
        ],
        out_specs=pl.BlockSpec(memory_space=pltpu.SMEM),
        out_shape=jax.ShapeDtypeStruct((1,), jnp.float32),
        scratch_shapes=[pltpu.VMEM((8, W), jnp.float32)],
    )(output, target)

    n = B * C * H * W
    return ssum[0] / jnp.float32(n)


# speedup vs baseline: 2.2720x; 2.1116x over previous
import jax
import jax.numpy as jnp
from jax.experimental import pallas as pl
from jax.experimental.pallas import tpu as pltpu


def _mse_onehot_kernel(x_ref, t_ref, out_ref, acc_ref):
    b = pl.program_id(0)
    h = pl.program_id(1)

    x = x_ref[0]
    t = t_ref[0]
    C, Hb, W = x.shape
    cidx = jax.lax.broadcasted_iota(jnp.int32, (C, Hb, W), 0)
    mask = (t[None, :, :] == cidx).astype(jnp.float32)
    d = x - mask
    d2 = d * d
    part = jnp.sum(d2.reshape(C * Hb // 8, 8, W), axis=0)

    first = jnp.logical_and(b == 0, h == 0)

    @pl.when(first)
    def _init():
        acc_ref[...] = part

    @pl.when(jnp.logical_not(first))
    def _accum():
        acc_ref[...] += part

    @pl.when(jnp.logical_and(b == pl.num_programs(0) - 1,
                             h == pl.num_programs(1) - 1))
    def _done():
        out_ref[0] = jnp.sum(acc_ref[...])


def kernel(output, target):
    B, C, H, W = output.shape
    target = target.astype(jnp.int32)

    HS = 2
    ssum = pl.pallas_call(
        _mse_onehot_kernel,
        grid=(B, HS),
        in_specs=[
            pl.BlockSpec((1, C, H // HS, W), lambda b, h: (b, 0, h, 0)),
            pl.BlockSpec((1, H // HS, W), lambda b, h: (b, h, 0)),
        ],
        out_specs=pl.BlockSpec(memory_space=pltpu.SMEM),
        out_shape=jax.ShapeDtypeStruct((1,), jnp.float32),
        scratch_shapes=[pltpu.VMEM((8, W), jnp.float32)],
    )(output, target)

    n = B * C * H * W
    return ssum[0] / jnp.float32(n)


# speedup vs baseline: 2.5197x; 1.1090x over previous
import jax
import jax.numpy as jnp
from jax.experimental import pallas as pl
from jax.experimental.pallas import tpu as pltpu


def _mse_onehot_kernel(x_ref, t_ref, out_ref, acc_ref):
    b = pl.program_id(0)
    h = pl.program_id(1)

    x = x_ref[0]
    t = t_ref[0]
    C, Hb, W = x.shape
    cidx = jax.lax.broadcasted_iota(jnp.int32, (C, Hb, W), 0)
    mask = (t[None, :, :] == cidx).astype(jnp.float32)
    d = x - mask
    d2 = (d * d).reshape(C * Hb, W)
    ones = jnp.ones((1, C * Hb), jnp.float32)
    part = jax.lax.dot_general(
        ones, d2, (((1,), (0,)), ((), ())),
        preferred_element_type=jnp.float32)

    first = jnp.logical_and(b == 0, h == 0)

    @pl.when(first)
    def _init():
        acc_ref[...] = part

    @pl.when(jnp.logical_not(first))
    def _accum():
        acc_ref[...] += part

    @pl.when(jnp.logical_and(b == pl.num_programs(0) - 1,
                             h == pl.num_programs(1) - 1))
    def _done():
        out_ref[0] = jnp.sum(acc_ref[...])


def kernel(output, target):
    B, C, H, W = output.shape
    target = target.astype(jnp.int32)

    HS = 2
    ssum = pl.pallas_call(
        _mse_onehot_kernel,
        grid=(B, HS),
        in_specs=[
            pl.BlockSpec((1, C, H // HS, W), lambda b, h: (b, 0, h, 0)),
            pl.BlockSpec((1, H // HS, W), lambda b, h: (b, h, 0)),
        ],
        out_specs=pl.BlockSpec(memory_space=pltpu.SMEM),
        out_shape=jax.ShapeDtypeStruct((1,), jnp.float32),
        scratch_shapes=[pltpu.VMEM((1, W), jnp.float32)],
    )(output, target)

    n = B * C * H * W
    return ssum[0] / jnp.float32(n)
